# baseline (speedup 1.0000x reference)
import functools
import jax
import jax.numpy as jnp
from jax import lax
from jax.experimental import pallas as pl
from jax.experimental.pallas import tpu as pltpu
from jax.experimental.pallas import tpu_sc as plsc

NEG_SLOPE = 0.3
NC, NS = 2, 16
NW = NC * NS
CHUNK = 500
DEPTH = 4
AHEAD = 2
L = 16


def _matmul_body(x_ref, w_ref, ab_ref):
    c_in = x_ref.shape[1]
    c_out = w_ref.shape[0]
    w = w_ref[...]
    w_loc, w_nb = w[:, :c_in], w[:, c_in:]
    w_cat = jnp.concatenate([w_loc - w_nb, w_nb], axis=0)
    y = lax.dot_general(x_ref[...], w_cat, (((1,), (1,)), ((), ())),
                        preferred_element_type=jnp.float32)
    ab_ref[0] = y[:, :c_out]
    ab_ref[1] = y[:, c_out:]


def _edge_body(ab_hbm, src_hbm, dst_hbm,
               s_out, c_out,
               s_sp, c_sp,
               src_v, dst_v, buf_v, ones_v, z32_v, z1_v,
               gsem, ssem, osem,
               *, n_pad, n_chunks):
    rows_per_tile = n_pad // NS
    zc = 128
    nz = rows_per_tile // zc
    c = lax.axis_index("c")
    s = lax.axis_index("s")
    wid = c * NS + s
    base = s * rows_per_tile
    sl = pl.ds(base, rows_per_tile)
    b_hbm = ab_hbm.at[1]

    pltpu.async_copy(src_hbm.at[wid], src_v, gsem.at[0])
    pltpu.async_copy(dst_hbm.at[wid], dst_v, gsem.at[1])

    def fill(i, carry):
        z32_v[i, pl.ds(0, L)] = jnp.zeros((L,), jnp.float32)
        z32_v[i, pl.ds(L, L)] = jnp.zeros((L,), jnp.float32)
        return carry
    lax.fori_loop(0, zc, fill, 0)
    for k in range(zc // L):
        z1_v[pl.ds(k * L, L)] = jnp.zeros((L,), jnp.float32)
    for k in range(CHUNK // L):
        ones_v[pl.ds(k * L, L)] = jnp.ones((L,), jnp.float32)

    for k in range(nz):
        pltpu.async_copy(z32_v, s_sp.at[pl.ds(base + k * zc, zc)],
                         osem.at[0])
        pltpu.async_copy(z1_v, c_sp.at[pl.ds(base + k * zc, zc)],
                         osem.at[1])
    for k in range(nz):
        pltpu.make_async_copy(z32_v, s_sp.at[pl.ds(base, zc)],
                              osem.at[0]).wait()
        pltpu.make_async_copy(z1_v, c_sp.at[pl.ds(base, zc)],
                              osem.at[1]).wait()
    pltpu.make_async_copy(src_hbm.at[wid], src_v, gsem.at[0]).wait()
    pltpu.make_async_copy(dst_hbm.at[wid], dst_v, gsem.at[1]).wait()
    plsc.subcore_barrier()

    def fire_gather(j, slot):
        pltpu.async_copy(b_hbm.at[src_v.at[j]], buf_v.at[slot], gsem.at[slot])

    def wait_gather(j, slot):
        pltpu.make_async_copy(b_hbm.at[src_v.at[j]], buf_v.at[slot],
                              gsem.at[slot]).wait()

    def fire_scatters(j, slot):
        pltpu.async_copy(buf_v.at[slot], s_sp.at[dst_v.at[j]], ssem.at[slot],
                         add=True)
        pltpu.async_copy(ones_v, c_sp.at[dst_v.at[j]], osem.at[slot], add=True)

    def wait_scatter(j, slot):
        pltpu.make_async_copy(buf_v.at[slot], s_sp.at[dst_v.at[j]],
                              ssem.at[slot]).wait()

    def wait_ones(j, slot):
        pltpu.make_async_copy(ones_v, c_sp.at[dst_v.at[j]],
                              osem.at[slot]).wait()

    for j in range(AHEAD):
        fire_gather(j, j % DEPTH)
    for j in range(AHEAD):
        wait_gather(j, j % DEPTH)
        fire_gather(j + AHEAD, (j + AHEAD) % DEPTH)
        fire_scatters(j, j % DEPTH)

    def body(j, carry):
        sg = lax.rem(j + AHEAD, DEPTH)
        wait_scatter(j - AHEAD, sg)
        fire_gather(j + AHEAD, sg)
        slot = lax.rem(j, DEPTH)
        wait_gather(j, slot)
        fire_scatters(j, slot)
        return carry

    lax.fori_loop(AHEAD, n_chunks - AHEAD, body, 0)

    for j in range(n_chunks - AHEAD, n_chunks):
        wait_scatter(j - AHEAD, (j + AHEAD) % DEPTH)
        wait_gather(j, j % DEPTH)
        fire_scatters(j, j % DEPTH)

    for j in range(n_chunks - AHEAD, n_chunks):
        wait_scatter(j, j % DEPTH)
    for slot in range(DEPTH):
        n_fired = len([k for k in range(n_chunks) if k % DEPTH == slot])
        for _ in range(n_fired):
            wait_ones(0, slot)

    plsc.subcore_barrier()

    pltpu.sync_copy(s_sp.at[sl], s_out.at[c, sl])
    pltpu.sync_copy(c_sp.at[sl], c_out.at[c, sl])


def _final_body(ab_hbm, bias_hbm, s_hbm, c_hbm, out_hbm,
                a_v, s0_v, s1_v, c0_v, c1_v, bias_v, out_v, fsem,
                *, n, c_out):
    rows = a_v.shape[0]
    c = lax.axis_index("c")
    s = lax.axis_index("s")
    wid = c * NS + s
    base = jnp.minimum(wid * rows, n - rows)
    sl = pl.ds(base, rows)

    pltpu.async_copy(ab_hbm.at[0, sl], a_v, fsem.at[0])
    pltpu.async_copy(s_hbm.at[0, sl], s0_v, fsem.at[1])
    pltpu.async_copy(s_hbm.at[1, sl], s1_v, fsem.at[2])
    pltpu.async_copy(c_hbm.at[0, sl], c0_v, fsem.at[3])
    pltpu.async_copy(c_hbm.at[1, sl], c1_v, fsem.at[4])
    pltpu.async_copy(bias_hbm, bias_v, fsem.at[5])
    pltpu.make_async_copy(ab_hbm.at[0, sl], a_v, fsem.at[0]).wait()
    pltpu.make_async_copy(s_hbm.at[0, sl], s0_v, fsem.at[1]).wait()
    pltpu.make_async_copy(s_hbm.at[1, sl], s1_v, fsem.at[2]).wait()
    pltpu.make_async_copy(c_hbm.at[0, sl], c0_v, fsem.at[3]).wait()
    pltpu.make_async_copy(c_hbm.at[1, sl], c1_v, fsem.at[4]).wait()
    pltpu.make_async_copy(bias_hbm, bias_v, fsem.at[5]).wait()

    n_half = c_out // L

    def row_block(rb, carry):
        rbase = rb * L
        cs = pl.ds(rbase, L)
        cnt16 = c0_v[cs] + c1_v[cs]
        inv16 = 1.0 / jnp.maximum(cnt16, 1.0)
        m16 = jnp.minimum(cnt16, 1.0)
        for rr in range(L):
            r = rbase + rr
            lane = jnp.full((L,), rr, jnp.int32)
            inv = jnp.take(inv16, lane)
            msk = jnp.take(m16, lane)
            for h in range(n_half):
                hs = pl.ds(h * L, L)
                a_h = a_v[r, hs] + bias_v[hs]
                s_h = s0_v[r, hs] + s1_v[r, hs]
                pre = a_h * msk + s_h * inv
                out_v[r, hs] = (jnp.maximum(pre, 0.0)
                                + NEG_SLOPE * jnp.minimum(pre, 0.0))
        return carry

    lax.fori_loop(0, rows // L, row_block, 0)
    pltpu.sync_copy(out_v, out_hbm.at[sl])


def kernel(features, neighborhood_source, neighborhood_target, W, b):
    n, c_in = features.shape
    e = neighborhood_source.shape[0]
    c_out = W.shape[0]

    n_pad = ((n + NW * 8 - 1) // (NW * 8)) * (NW * 8)
    rows_f = n_pad // NW
    assert e % (NW * CHUNK) == 0
    n_chunks = e // (NW * CHUNK)

    src_r = neighborhood_source.reshape(NW, n_chunks, CHUNK)
    dst_r = neighborhood_target.reshape(NW, n_chunks, CHUNK)

    ab = pl.pallas_call(
        _matmul_body,
        out_shape=jax.ShapeDtypeStruct((2, n, c_out), jnp.float32),
    )(features, W)

    mesh = plsc.VectorSubcoreMesh(core_axis_name="c", subcore_axis_name="s",
                                  num_cores=NC, num_subcores=NS)
    edge_kernel = pl.kernel(
        functools.partial(_edge_body, n_pad=n_pad, n_chunks=n_chunks),
        out_type=(jax.ShapeDtypeStruct((NC, n_pad, c_out), jnp.float32),
                  jax.ShapeDtypeStruct((NC, n_pad), jnp.float32)),
        mesh=mesh,
        scratch_types=[
            pltpu.VMEM_SHARED((n_pad, c_out), jnp.float32),
            pltpu.VMEM_SHARED((n_pad,), jnp.float32),
            pltpu.VMEM((n_chunks, CHUNK), jnp.int32),
            pltpu.VMEM((n_chunks, CHUNK), jnp.int32),
            pltpu.VMEM((DEPTH, CHUNK, c_out), jnp.float32),
            pltpu.VMEM((CHUNK,), jnp.float32),
            pltpu.VMEM((128, c_out), jnp.float32),
            pltpu.VMEM((128,), jnp.float32),
            pltpu.SemaphoreType.DMA((DEPTH,)),
            pltpu.SemaphoreType.DMA((DEPTH,)),
            pltpu.SemaphoreType.DMA((DEPTH,)),
        ],
        compiler_params=pltpu.CompilerParams(use_tc_tiling_on_sc=False),
    )
    s_part, c_part = edge_kernel(ab, src_r, dst_r)

    final_kernel = pl.kernel(
        functools.partial(_final_body, n=n, c_out=c_out),
        out_type=jax.ShapeDtypeStruct((n, c_out), jnp.float32),
        mesh=plsc.VectorSubcoreMesh(core_axis_name="c", subcore_axis_name="s",
                                    num_cores=NC, num_subcores=NS),
        scratch_types=[
            pltpu.VMEM((rows_f, c_out), jnp.float32),
            pltpu.VMEM((rows_f, c_out), jnp.float32),
            pltpu.VMEM((rows_f, c_out), jnp.float32),
            pltpu.VMEM((rows_f,), jnp.float32),
            pltpu.VMEM((rows_f,), jnp.float32),
            pltpu.VMEM((c_out,), jnp.float32),
            pltpu.VMEM((rows_f, c_out), jnp.float32),
            pltpu.SemaphoreType.DMA((6,)),
        ],
        compiler_params=pltpu.CompilerParams(use_tc_tiling_on_sc=False),
    )
    return final_kernel(ab, b, s_part, c_part)


# speedup vs baseline: 1.1416x; 1.1416x over previous
import functools
import jax
import jax.numpy as jnp
from jax import lax
from jax.experimental import pallas as pl
from jax.experimental.pallas import tpu as pltpu
from jax.experimental.pallas import tpu_sc as plsc

NEG_SLOPE = 0.3
NC, NS = 2, 16
NW = NC * NS
CHUNK = 80
DEPTH = 6
AHEAD = 3
L = 16


def _matmul_body(x_ref, w_ref, ab_ref):
    c_in = x_ref.shape[1]
    c_out = w_ref.shape[0]
    w = w_ref[...]
    w_loc, w_nb = w[:, :c_in], w[:, c_in:]
    w_cat = jnp.concatenate([w_loc - w_nb, w_nb], axis=0)
    y = lax.dot_general(x_ref[...], w_cat, (((1,), (1,)), ((), ())),
                        preferred_element_type=jnp.float32)
    ab_ref[0] = y[:, :c_out].astype(jnp.bfloat16)
    ab_ref[1] = y[:, c_out:].astype(jnp.bfloat16)


def _edge_body(ab_hbm, src_hbm, dst_hbm,
               s_out, c_out,
               s_sp, c_sp,
               src_v, dst_v, buf_v, ones_v, z32_v, z1_v,
               gsem, ssem, osem,
               *, n_pad, n_chunks):
    rows_per_tile = n_pad // NS
    zc = 128
    nz = rows_per_tile // zc
    c = lax.axis_index("c")
    s = lax.axis_index("s")
    wid = c * NS + s
    base = s * rows_per_tile
    sl = pl.ds(base, rows_per_tile)
    b_hbm = ab_hbm.at[1]

    pltpu.async_copy(src_hbm.at[wid], src_v, gsem.at[0])
    pltpu.async_copy(dst_hbm.at[wid], dst_v, gsem.at[1])

    def fill(i, carry):
        z32_v[i, :] = jnp.zeros((2 * L,), jnp.bfloat16)
        return carry
    lax.fori_loop(0, zc, fill, 0)
    for k in range(zc // L):
        z1_v[pl.ds(k * L, L)] = jnp.zeros((L,), jnp.float32)
    for k in range(CHUNK // L):
        ones_v[pl.ds(k * L, L)] = jnp.ones((L,), jnp.float32)

    for k in range(nz):
        pltpu.async_copy(z32_v, s_sp.at[pl.ds(base + k * zc, zc)],
                         osem.at[0])
        pltpu.async_copy(z1_v, c_sp.at[pl.ds(base + k * zc, zc)],
                         osem.at[1])
    for k in range(nz):
        pltpu.make_async_copy(z32_v, s_sp.at[pl.ds(base, zc)],
                              osem.at[0]).wait()
        pltpu.make_async_copy(z1_v, c_sp.at[pl.ds(base, zc)],
                              osem.at[1]).wait()
    pltpu.make_async_copy(src_hbm.at[wid], src_v, gsem.at[0]).wait()
    pltpu.make_async_copy(dst_hbm.at[wid], dst_v, gsem.at[1]).wait()
    plsc.subcore_barrier()

    def fire_gather(j, slot):
        pltpu.async_copy(b_hbm.at[src_v.at[j]], buf_v.at[slot], gsem.at[slot])

    def wait_gather(j, slot):
        pltpu.make_async_copy(b_hbm.at[src_v.at[j]], buf_v.at[slot],
                              gsem.at[slot]).wait()

    def fire_scatters(j, slot):
        pltpu.async_copy(buf_v.at[slot], s_sp.at[dst_v.at[j]], ssem.at[slot],
                         add=True)
        pltpu.async_copy(ones_v, c_sp.at[dst_v.at[j]], osem.at[slot], add=True)

    def wait_scatter(j, slot):
        pltpu.make_async_copy(buf_v.at[slot], s_sp.at[dst_v.at[j]],
                              ssem.at[slot]).wait()

    def wait_ones(j, slot):
        pltpu.make_async_copy(ones_v, c_sp.at[dst_v.at[j]],
                              osem.at[slot]).wait()

    for j in range(AHEAD):
        fire_gather(j, j % DEPTH)
    for j in range(AHEAD):
        wait_gather(j, j % DEPTH)
        fire_gather(j + AHEAD, (j + AHEAD) % DEPTH)
        fire_scatters(j, j % DEPTH)

    def body(j, carry):
        sg = lax.rem(j + AHEAD, DEPTH)
        wait_scatter(j - AHEAD, sg)
        fire_gather(j + AHEAD, sg)
        slot = lax.rem(j, DEPTH)
        wait_gather(j, slot)
        fire_scatters(j, slot)
        return carry

    lax.fori_loop(AHEAD, n_chunks - AHEAD, body, 0)

    for j in range(n_chunks - AHEAD, n_chunks):
        wait_scatter(j - AHEAD, (j + AHEAD) % DEPTH)
        wait_gather(j, j % DEPTH)
        fire_scatters(j, j % DEPTH)

    for j in range(n_chunks - AHEAD, n_chunks):
        wait_scatter(j, j % DEPTH)
    for slot in range(DEPTH):
        n_fired = len([k for k in range(n_chunks) if k % DEPTH == slot])
        for _ in range(n_fired):
            wait_ones(0, slot)

    plsc.subcore_barrier()

    pltpu.sync_copy(s_sp.at[sl], s_out.at[c, sl])
    pltpu.sync_copy(c_sp.at[sl], c_out.at[c, sl])


def _final_body(ab_hbm, bias_hbm, s_hbm, c_hbm, out_hbm,
                a_v, s0_v, s1_v, c0_v, c1_v, bias_v, out_v, fsem,
                *, n, c_out):
    rows = a_v.shape[0]
    c = lax.axis_index("c")
    s = lax.axis_index("s")
    wid = c * NS + s
    base = jnp.minimum(wid * rows, n - rows)
    sl = pl.ds(base, rows)

    pltpu.async_copy(ab_hbm.at[0, sl], a_v, fsem.at[0])
    pltpu.async_copy(s_hbm.at[0, sl], s0_v, fsem.at[1])
    pltpu.async_copy(s_hbm.at[1, sl], s1_v, fsem.at[2])
    pltpu.async_copy(c_hbm.at[0, sl], c0_v, fsem.at[3])
    pltpu.async_copy(c_hbm.at[1, sl], c1_v, fsem.at[4])
    pltpu.async_copy(bias_hbm, bias_v, fsem.at[5])
    pltpu.make_async_copy(ab_hbm.at[0, sl], a_v, fsem.at[0]).wait()
    pltpu.make_async_copy(s_hbm.at[0, sl], s0_v, fsem.at[1]).wait()
    pltpu.make_async_copy(s_hbm.at[1, sl], s1_v, fsem.at[2]).wait()
    pltpu.make_async_copy(c_hbm.at[0, sl], c0_v, fsem.at[3]).wait()
    pltpu.make_async_copy(c_hbm.at[1, sl], c1_v, fsem.at[4]).wait()
    pltpu.make_async_copy(bias_hbm, bias_v, fsem.at[5]).wait()

    zero_b = jnp.zeros((2 * L,), jnp.bfloat16)
    slope_b = jnp.full((2 * L,), NEG_SLOPE, jnp.bfloat16)

    def row_block(rb, carry):
        rbase = rb * L
        cs = pl.ds(rbase, L)
        cnt16 = c0_v[cs] + c1_v[cs]
        inv16 = 1.0 / jnp.maximum(cnt16, 1.0)
        m16 = jnp.minimum(cnt16, 1.0)
        for rr in range(L):
            r = rbase + rr
            lane = jnp.full((L,), rr, jnp.int32)
            inv = jnp.take(inv16, lane)
            msk = jnp.take(m16, lane)
            inv_b = plsc.pack(inv, inv, format=plsc.PackFormat.INTERLEAVED)
            msk_b = plsc.pack(msk, msk, format=plsc.PackFormat.INTERLEAVED)
            a_row = a_v[r, :] + bias_v[...]
            s_row = s0_v[r, :] + s1_v[r, :]
            pre = a_row * msk_b + s_row * inv_b
            out_v[r, :] = (jnp.maximum(pre, zero_b)
                           + slope_b * jnp.minimum(pre, zero_b))
        return carry

    lax.fori_loop(0, rows // L, row_block, 0)
    pltpu.sync_copy(out_v, out_hbm.at[sl])


def kernel(features, neighborhood_source, neighborhood_target, W, b):
    n, c_in = features.shape
    e = neighborhood_source.shape[0]
    c_out = W.shape[0]

    n_pad = ((n + NW * 8 - 1) // (NW * 8)) * (NW * 8)
    rows_f = n_pad // NW
    assert e % (NW * CHUNK) == 0
    n_chunks = e // (NW * CHUNK)

    src_r = neighborhood_source.reshape(NW, n_chunks, CHUNK)
    dst_r = neighborhood_target.reshape(NW, n_chunks, CHUNK)

    ab = pl.pallas_call(
        _matmul_body,
        out_shape=jax.ShapeDtypeStruct((2, n, c_out), jnp.bfloat16),
    )(features, W)

    mesh = plsc.VectorSubcoreMesh(core_axis_name="c", subcore_axis_name="s",
                                  num_cores=NC, num_subcores=NS)
    edge_kernel = pl.kernel(
        functools.partial(_edge_body, n_pad=n_pad, n_chunks=n_chunks),
        out_type=(jax.ShapeDtypeStruct((NC, n_pad, c_out), jnp.bfloat16),
                  jax.ShapeDtypeStruct((NC, n_pad), jnp.float32)),
        mesh=mesh,
        scratch_types=[
            pltpu.VMEM_SHARED((n_pad, c_out), jnp.bfloat16),
            pltpu.VMEM_SHARED((n_pad,), jnp.float32),
            pltpu.VMEM((n_chunks, CHUNK), jnp.int32),
            pltpu.VMEM((n_chunks, CHUNK), jnp.int32),
            pltpu.VMEM((DEPTH, CHUNK, c_out), jnp.bfloat16),
            pltpu.VMEM((CHUNK,), jnp.float32),
            pltpu.VMEM((128, c_out), jnp.bfloat16),
            pltpu.VMEM((128,), jnp.float32),
            pltpu.SemaphoreType.DMA((DEPTH,)),
            pltpu.SemaphoreType.DMA((DEPTH,)),
            pltpu.SemaphoreType.DMA((DEPTH,)),
        ],
        compiler_params=pltpu.CompilerParams(use_tc_tiling_on_sc=False,
                                             needs_layout_passes=False),
    )
    s_part, c_part = edge_kernel(ab, src_r, dst_r)

    final_kernel = pl.kernel(
        functools.partial(_final_body, n=n, c_out=c_out),
        out_type=jax.ShapeDtypeStruct((n, c_out), jnp.bfloat16),
        mesh=plsc.VectorSubcoreMesh(core_axis_name="c", subcore_axis_name="s",
                                    num_cores=NC, num_subcores=NS),
        scratch_types=[
            pltpu.VMEM((rows_f, c_out), jnp.bfloat16),
            pltpu.VMEM((rows_f, c_out), jnp.bfloat16),
            pltpu.VMEM((rows_f, c_out), jnp.bfloat16),
            pltpu.VMEM((rows_f,), jnp.float32),
            pltpu.VMEM((rows_f,), jnp.float32),
            pltpu.VMEM((c_out,), jnp.bfloat16),
            pltpu.VMEM((rows_f, c_out), jnp.bfloat16),
            pltpu.SemaphoreType.DMA((6,)),
        ],
        compiler_params=pltpu.CompilerParams(use_tc_tiling_on_sc=False,
                                             needs_layout_passes=False),
    )
    out_bf = final_kernel(ab, b.astype(jnp.bfloat16), s_part, c_part)
    return out_bf.astype(jnp.float32)
